# baseline (speedup 1.0000x reference)
import functools

import jax
import jax.numpy as jnp
from jax import lax
from jax.experimental import pallas as pl
from jax.experimental.pallas import tpu as pltpu
from jax.experimental.pallas import tpu_sc as plsc

_B = 4096 * 26
_D = 64
_NW = 32
_ROWS_PER_W = _B // _NW
_NCHUNK = 8
_CHUNK = _ROWS_PER_W // _NCHUNK
_NG = 4
_GL = _CHUNK // _NG


def _emb_body(idx_hbm, table_hbm, out_hbm, idx_v, gidx_v, rows0, rows1,
              g0s, g1s, w0s, w1s):
    wid = lax.axis_index("s") * 2 + lax.axis_index("c")
    base = wid * _ROWS_PER_W
    pltpu.sync_copy(idx_hbm.at[pl.ds(base, _ROWS_PER_W)], idx_v)

    def _shift(i, carry):
        r = i * 16
        gidx_v[pl.ds(r, 16)] = lax.shift_right_logical(idx_v[pl.ds(r, 16)], 1)
        return carry

    lax.fori_loop(0, _ROWS_PER_W // 16, _shift, 0)

    bufs = (rows0, rows1)
    gsems = (g0s, g1s)
    wsems = (w0s, w1s)

    def fire(c):
        b = c & 1
        return [
            pltpu.async_copy(
                table_hbm.at[gidx_v.at[pl.ds(c * _CHUNK + g * _GL, _GL)]],
                bufs[b].at[pl.ds(g * _GL, _GL)],
                gsems[b],
            )
            for g in range(_NG)
        ]

    writes = [None] * _NCHUNK
    gathers = [None] * _NCHUNK
    gathers[0] = fire(0)
    for c in range(_NCHUNK):
        b = c & 1
        if c + 1 < _NCHUNK:
            if c >= 1:
                writes[c - 1].wait()
            gathers[c + 1] = fire(c + 1)
        for cp in gathers[c]:
            cp.wait()

        def _fix(j, carry):
            ij = plsc.load_gather(
                idx_v, [jnp.full((16,), c * _CHUNK + j, jnp.int32)]
            )
            odd = (ij & 1) == 1
            zero = ij == 0
            for q in range(4):
                a = bufs[b][j, pl.ds(q * 16, 16)]
                hi = bufs[b][j, pl.ds(64 + q * 16, 16)]
                v = jnp.where(odd, hi, a)
                v = jnp.where(zero, 0.0, v)
                bufs[b][j, pl.ds(q * 16, 16)] = v
            return carry

        lax.fori_loop(0, _CHUNK, _fix, 0)
        writes[c] = pltpu.async_copy(
            bufs[b], out_hbm.at[pl.ds(base + c * _CHUNK, _CHUNK)], wsems[b]
        )
    writes[_NCHUNK - 2].wait()
    writes[_NCHUNK - 1].wait()


_emb = functools.partial(
    pl.kernel,
    out_type=jax.ShapeDtypeStruct((_B, 2 * _D), jnp.float32),
    mesh=plsc.VectorSubcoreMesh(core_axis_name="c", subcore_axis_name="s"),
    compiler_params=pltpu.CompilerParams(
        needs_layout_passes=False, use_tc_tiling_on_sc=True
    ),
    scratch_types=[
        pltpu.VMEM((_ROWS_PER_W,), jnp.int32),
        pltpu.VMEM((_ROWS_PER_W,), jnp.int32),
        pltpu.VMEM((_CHUNK, 2 * _D), jnp.float32),
        pltpu.VMEM((_CHUNK, 2 * _D), jnp.float32),
        pltpu.SemaphoreType.DMA,
        pltpu.SemaphoreType.DMA,
        pltpu.SemaphoreType.DMA,
        pltpu.SemaphoreType.DMA,
    ],
)(_emb_body)


def kernel(input, table):
    idx_flat = input.reshape(_B)
    pairs = table.reshape(table.shape[0] // 2, 2 * _D)
    out = _emb(idx_flat, pairs)
    return out[:, :_D].reshape(input.shape[0], input.shape[1], _D)


# speedup vs baseline: 1.0317x; 1.0317x over previous
import functools

import jax
import jax.numpy as jnp
from jax import lax
from jax.experimental import pallas as pl
from jax.experimental.pallas import tpu as pltpu
from jax.experimental.pallas import tpu_sc as plsc

_B = 4096 * 26
_D = 64
_NW = 32
_ROWS_PER_W = _B // _NW
_NCHUNK = 4
_CHUNK = _ROWS_PER_W // _NCHUNK
_NG = 8
_GL = _CHUNK // _NG


def _emb_body(idx_hbm, table_hbm, out_hbm, idx_v, rows0, rows1, g0s, g1s,
              w0s, w1s):
    wid = lax.axis_index("s") * 2 + lax.axis_index("c")
    base = wid * _ROWS_PER_W
    pltpu.sync_copy(idx_hbm.at[pl.ds(base, _ROWS_PER_W)], idx_v)
    bufs = (rows0, rows1)
    gsems = (g0s, g1s)
    wsems = (w0s, w1s)

    def fire(c):
        b = c & 1
        return [
            pltpu.async_copy(
                table_hbm.at[idx_v.at[pl.ds(c * _CHUNK + g * _GL, _GL)]],
                bufs[b].at[pl.ds(g * _GL, _GL)],
                gsems[b],
            )
            for g in range(_NG)
        ]

    writes = [None] * _NCHUNK
    gathers = [None] * _NCHUNK
    gathers[0] = fire(0)
    for c in range(_NCHUNK):
        b = c & 1
        if c + 1 < _NCHUNK:
            if c >= 1:
                writes[c - 1].wait()
            gathers[c + 1] = fire(c + 1)
        for cp in gathers[c]:
            cp.wait()

        def _mask_fix(i, carry):
            r = i * 16
            idxs = idx_v[pl.ds(c * _CHUNK + r, 16)]

            @pl.when(jnp.any(idxs == 0))
            def _():
                for j in range(16):
                    ij = plsc.load_gather(
                        idx_v, [jnp.full((16,), c * _CHUNK + r + j, jnp.int32)]
                    )
                    mj = jnp.where(ij == 0, 0.0, 1.0).astype(jnp.float32)
                    for q in range(4):
                        sl = (r + j, pl.ds(q * 16, 16))
                        bufs[b][sl] = bufs[b][sl] * mj

            return carry

        lax.fori_loop(0, _CHUNK // 16, _mask_fix, 0)
        writes[c] = pltpu.async_copy(
            bufs[b],
            out_hbm.at[pl.ds(base + c * _CHUNK, _CHUNK), pl.ds(0, _D)],
            wsems[b],
        )
    writes[_NCHUNK - 2].wait()
    writes[_NCHUNK - 1].wait()


_emb = functools.partial(
    pl.kernel,
    out_type=jax.ShapeDtypeStruct((_B, 2 * _D), jnp.float32),
    mesh=plsc.VectorSubcoreMesh(core_axis_name="c", subcore_axis_name="s"),
    compiler_params=pltpu.CompilerParams(
        needs_layout_passes=False, use_tc_tiling_on_sc=False
    ),
    scratch_types=[
        pltpu.VMEM((_ROWS_PER_W,), jnp.int32),
        pltpu.VMEM((_CHUNK, _D), jnp.float32),
        pltpu.VMEM((_CHUNK, _D), jnp.float32),
        pltpu.SemaphoreType.DMA,
        pltpu.SemaphoreType.DMA,
        pltpu.SemaphoreType.DMA,
        pltpu.SemaphoreType.DMA,
    ],
)(_emb_body)


def kernel(input, table):
    idx_flat = input.reshape(_B)
    out = _emb(idx_flat, table)
    return out[:, :_D].reshape(input.shape[0], input.shape[1], _D)


# speedup vs baseline: 1.0500x; 1.0177x over previous
import functools

import jax
import jax.numpy as jnp
from jax import lax
from jax.experimental import pallas as pl
from jax.experimental.pallas import tpu as pltpu
from jax.experimental.pallas import tpu_sc as plsc

_B = 4096 * 26
_D = 64
_NW = 32
_ROWS_PER_W = _B // _NW
_NCHUNK = 4
_CHUNK = _ROWS_PER_W // _NCHUNK
_NG = 8
_GL = _CHUNK // _NG


def _emb_body(idx_hbm, table_hbm, out_hbm, idx_v, rows0, rows1, g0s, g1s,
              w0s, w1s):
    wid = lax.axis_index("s") * 2 + lax.axis_index("c")
    base = wid * _ROWS_PER_W
    pltpu.sync_copy(idx_hbm.at[pl.ds(base, _ROWS_PER_W)], idx_v)
    bufs = (rows0, rows1)
    gsems = (g0s, g1s)
    wsems = (w0s, w1s)

    def fire(c):
        b = c & 1
        return [
            pltpu.async_copy(
                table_hbm.at[idx_v.at[pl.ds(c * _CHUNK + g * _GL, _GL)]],
                bufs[b].at[pl.ds(g * _GL, _GL)],
                gsems[b],
            )
            for g in range(_NG)
        ]

    writes = [None] * _NCHUNK
    gathers = [None] * _NCHUNK
    gathers[0] = fire(0)
    for c in range(_NCHUNK):
        b = c & 1
        if c + 1 < _NCHUNK:
            if c >= 1:
                writes[c - 1].wait()
            gathers[c + 1] = fire(c + 1)
        for cp in gathers[c]:
            cp.wait()

        def _mask_fix(i, carry):
            r = i * 16
            idxs = idx_v[pl.ds(c * _CHUNK + r, 16)]

            @pl.when(jnp.any(idxs == 0))
            def _():
                for j in range(16):
                    ij = plsc.load_gather(
                        idx_v, [jnp.full((16,), c * _CHUNK + r + j, jnp.int32)]
                    )
                    mj = jnp.where(ij == 0, 0.0, 1.0).astype(jnp.float32)
                    for q in range(4):
                        sl = (r + j, pl.ds(q * 16, 16))
                        bufs[b][sl] = bufs[b][sl] * mj

            return carry

        lax.fori_loop(0, _CHUNK // 16, _mask_fix, 0)
        writes[c] = pltpu.async_copy(
            bufs[b], out_hbm.at[pl.ds(base + c * _CHUNK, _CHUNK)], wsems[b]
        )
    writes[_NCHUNK - 2].wait()
    writes[_NCHUNK - 1].wait()


_emb = functools.partial(
    pl.kernel,
    out_type=jax.ShapeDtypeStruct((_B, _D), jnp.float32),
    mesh=plsc.VectorSubcoreMesh(core_axis_name="c", subcore_axis_name="s"),
    compiler_params=pltpu.CompilerParams(
        needs_layout_passes=False, use_tc_tiling_on_sc=False
    ),
    scratch_types=[
        pltpu.VMEM((_ROWS_PER_W,), jnp.int32),
        pltpu.VMEM((_CHUNK, _D), jnp.float32),
        pltpu.VMEM((_CHUNK, _D), jnp.float32),
        pltpu.SemaphoreType.DMA,
        pltpu.SemaphoreType.DMA,
        pltpu.SemaphoreType.DMA,
        pltpu.SemaphoreType.DMA,
    ],
)(_emb_body)


def kernel(input, table):
    idx_flat = input.reshape(_B)
    out = _emb(idx_flat, table)
    return out.reshape(input.shape[0], input.shape[1], _D)
